# baseline (speedup 1.0000x reference)
import functools

import jax
import jax.numpy as jnp
from jax import lax
from jax.experimental import pallas as pl
from jax.experimental.pallas import tpu as pltpu
from jax.experimental.pallas import tpu_sc as plsc

D_OUT = 64
SCALE = 8.0
LANES = 16
SUB = 128
SUBS_PER_MACRO = 4
MACRO = SUB * SUBS_PER_MACRO


@functools.cache
def _build(B: int):
    info = plsc.get_sparse_core_info()
    NC, NS = info.num_cores, info.num_subcores
    NW = NC * NS
    assert B % (NW * MACRO) == 0, (B, NW, MACRO)
    b_per_w = B // NW
    n_macro = b_per_w // MACRO
    assert n_macro % 2 == 0
    n_pair = n_macro // 2
    mesh = plsc.VectorSubcoreMesh(core_axis_name="c", subcore_axis_name="s")

    def body(x_hbm, table_hbm, out_hbm, idx_v, rows_a, rows_b, sem_a, sem_b):
        wid = lax.axis_index("s") * NC + lax.axis_index("c")
        base = wid * b_per_w
        pltpu.sync_copy(x_hbm.at[pl.ds(base, b_per_w)], idx_v)

        def fire(m, buf, sem):
            for j in range(SUBS_PER_MACRO):
                pltpu.async_copy(
                    table_hbm.at[idx_v.at[pl.ds(m * MACRO + j * SUB, SUB)]],
                    buf.at[pl.ds(j * SUB, SUB)],
                    sem,
                )

        def drain(m, buf, sem):
            for j in range(SUBS_PER_MACRO):
                pltpu.make_async_copy(
                    table_hbm.at[idx_v.at[pl.ds(m * MACRO + j * SUB, SUB)]],
                    buf.at[pl.ds(j * SUB, SUB)],
                    sem,
                ).wait()

        def scale_out(m, buf):
            def row(r, carry):
                for jj in range(D_OUT // LANES):
                    sl = pl.ds(jj * LANES, LANES)
                    buf[r, sl] = buf[r, sl] * SCALE
                return carry

            lax.fori_loop(0, MACRO, row, 0)
            pltpu.sync_copy(buf, out_hbm.at[pl.ds(base + m * MACRO, MACRO)])

        fire(0, rows_a, sem_a)

        def pair(g, carry):
            c0 = 2 * g
            fire(c0 + 1, rows_b, sem_b)
            drain(c0, rows_a, sem_a)
            scale_out(c0, rows_a)

            @pl.when(g + 1 < n_pair)
            def _():
                fire(c0 + 2, rows_a, sem_a)

            drain(c0 + 1, rows_b, sem_b)
            scale_out(c0 + 1, rows_b)
            return carry

        lax.fori_loop(0, n_pair, pair, 0)

    return pl.kernel(
        body,
        out_type=jax.ShapeDtypeStruct((B, D_OUT), jnp.float32),
        mesh=mesh,
        compiler_params=pltpu.CompilerParams(use_tc_tiling_on_sc=False),
        scratch_types=[
            pltpu.VMEM((b_per_w,), jnp.int32),
            pltpu.VMEM((MACRO, D_OUT), jnp.float32),
            pltpu.VMEM((MACRO, D_OUT), jnp.float32),
            pltpu.SemaphoreType.DMA,
            pltpu.SemaphoreType.DMA,
        ],
    )


def kernel(x, table):
    s0, s1 = x.shape
    B = s0 * s1
    flat = x.reshape(B).astype(jnp.int32)
    out = _build(B)(flat, table)
    return out.reshape(s0, s1, D_OUT)


# speedup vs baseline: 1.1276x; 1.1276x over previous
import functools

import jax
import jax.numpy as jnp
from jax import lax
from jax.experimental import pallas as pl
from jax.experimental.pallas import tpu as pltpu
from jax.experimental.pallas import tpu_sc as plsc

D_OUT = 64
SCALE = 8.0
LANES = 16
BLK = 128
TPAD = 129


@functools.cache
def _build(S0: int, S1: int, V: int):
    info = plsc.get_sparse_core_info()
    NC, NS = info.num_cores, info.num_subcores
    NW = NC * NS
    s0_blocks = S0 // BLK
    n_blocks = S1 * s0_blocks
    assert n_blocks % (2 * NW) == 0, (S0, S1, NW)
    blocks_per_w = n_blocks // NW
    n_pair = blocks_per_w // 2
    b_per_w = blocks_per_w * BLK
    mesh = plsc.VectorSubcoreMesh(core_axis_name="c", subcore_axis_name="s")

    def body(x_hbm, table_hbm, out_hbm, idx_v, rows_a, rows_b,
             trans_a, trans_b, sem_a, sem_b, sem_oa, sem_ob):
        wid = lax.axis_index("s") * NC + lax.axis_index("c")
        base = wid * b_per_w
        pltpu.sync_copy(x_hbm.at[pl.ds(base, b_per_w)], idx_v)
        iota16 = lax.iota(jnp.int32, 16)
        rowv = [iota16 + j0 for j0 in range(0, D_OUT, LANES)]

        def fire_gather(b, buf, sem):
            pltpu.async_copy(
                table_hbm.at[idx_v.at[pl.ds(b * BLK, BLK)]], buf, sem
            )

        def drain_gather(b, buf, sem):
            pltpu.make_async_copy(
                table_hbm.at[idx_v.at[pl.ds(b * BLK, BLK)]], buf, sem
            ).wait()

        def out_tiles(b, trans, sem, fire):
            blk_id = wid * blocks_per_w + b
            s1 = blk_id // s0_blocks
            s0b = lax.rem(blk_id, s0_blocks)
            for jb in range(D_OUT // 8):
                src = trans.at[pl.ds(jb * 8, 8), pl.ds(0, BLK)]
                dst = out_hbm.at[s1, jb, s0b]
                if fire:
                    pltpu.async_copy(src, dst, sem)
                else:
                    pltpu.make_async_copy(src, dst, sem).wait()

        def transpose_scale(rows, trans):
            def tok(t, carry):
                colv = jnp.full((LANES,), 0, jnp.int32) + t
                for k, j0 in enumerate(range(0, D_OUT, LANES)):
                    v = rows[t, pl.ds(j0, LANES)] * SCALE
                    plsc.store_scatter(trans, [rowv[k], colv], v)
                return carry

            lax.fori_loop(0, BLK, tok, 0)

        fire_gather(0, rows_a, sem_a)

        def pair(g, carry):
            c0 = 2 * g
            fire_gather(c0 + 1, rows_b, sem_b)
            drain_gather(c0, rows_a, sem_a)

            @pl.when(g > 0)
            def _():
                out_tiles(c0 - 2, trans_a, sem_oa, False)

            transpose_scale(rows_a, trans_a)
            out_tiles(c0, trans_a, sem_oa, True)

            @pl.when(g + 1 < n_pair)
            def _():
                fire_gather(c0 + 2, rows_a, sem_a)

            drain_gather(c0 + 1, rows_b, sem_b)

            @pl.when(g > 0)
            def _():
                out_tiles(c0 - 1, trans_b, sem_ob, False)

            transpose_scale(rows_b, trans_b)
            out_tiles(c0 + 1, trans_b, sem_ob, True)
            return carry

        lax.fori_loop(0, n_pair, pair, 0)
        out_tiles(blocks_per_w - 2, trans_a, sem_oa, False)
        out_tiles(blocks_per_w - 1, trans_b, sem_ob, False)

    return pl.kernel(
        body,
        out_type=jax.ShapeDtypeStruct(
            (S1, D_OUT // 8, S0 // BLK, 8, BLK), jnp.float32
        ),
        mesh=mesh,
        compiler_params=pltpu.CompilerParams(
            use_tc_tiling_on_sc=False, needs_layout_passes=False
        ),
        scratch_types=[
            pltpu.VMEM((b_per_w,), jnp.int32),
            pltpu.VMEM((BLK, D_OUT), jnp.float32),
            pltpu.VMEM((BLK, D_OUT), jnp.float32),
            pltpu.VMEM((D_OUT, TPAD), jnp.float32),
            pltpu.VMEM((D_OUT, TPAD), jnp.float32),
            pltpu.SemaphoreType.DMA,
            pltpu.SemaphoreType.DMA,
            pltpu.SemaphoreType.DMA,
            pltpu.SemaphoreType.DMA,
        ],
    )


def kernel(x, table):
    s0, s1 = x.shape
    v, d = table.shape
    xt = x.T.reshape(s0 * s1).astype(jnp.int32)
    out5d = _build(s0, s1, v)(xt, table)
    return out5d.transpose(2, 4, 0, 1, 3).reshape(s0, s1, d)


# speedup vs baseline: 1.1458x; 1.0162x over previous
import functools

import jax
import jax.numpy as jnp
from jax import lax
from jax.experimental import pallas as pl
from jax.experimental.pallas import tpu as pltpu
from jax.experimental.pallas import tpu_sc as plsc

D_OUT = 64
SCALE = 8.0
LANES = 16
BLK = 128
TPAD = 129


@functools.cache
def _build(S0: int, S1: int, V: int):
    info = plsc.get_sparse_core_info()
    NC, NS = info.num_cores, info.num_subcores
    NW = NC * NS
    s0_blocks = S0 // BLK
    n_blocks = S1 * s0_blocks
    assert n_blocks % (2 * NW) == 0, (S0, S1, NW)
    blocks_per_w = n_blocks // NW
    n_pair = blocks_per_w // 2
    b_per_w = blocks_per_w * BLK
    mesh = plsc.VectorSubcoreMesh(core_axis_name="c", subcore_axis_name="s")

    def body(x_hbm, table_hbm, out_hbm, idx_v, rows_a, rows_b,
             trans_a, trans_b, sem_a, sem_b, sem_oa, sem_ob):
        wid = lax.axis_index("s") * NC + lax.axis_index("c")
        base = wid * b_per_w
        pltpu.sync_copy(x_hbm.at[pl.ds(base, b_per_w)], idx_v)
        iota16 = lax.iota(jnp.int32, 16)
        rowv = [iota16 + j0 for j0 in range(0, D_OUT, LANES)]

        def fire_gather(b, buf, sem):
            pltpu.async_copy(
                table_hbm.at[idx_v.at[pl.ds(b * BLK, BLK)]], buf, sem
            )

        def drain_gather(b, buf, sem):
            pltpu.make_async_copy(
                table_hbm.at[idx_v.at[pl.ds(b * BLK, BLK)]], buf, sem
            ).wait()

        def out_tiles(b, trans, sem, fire):
            blk_id = wid * blocks_per_w + b
            s1 = blk_id // s0_blocks
            s0b = lax.rem(blk_id, s0_blocks)
            for jb in range(D_OUT // 8):
                src = trans.at[pl.ds(jb * 8, 8), pl.ds(0, BLK)]
                dst = out_hbm.at[s1, jb, s0b]
                if fire:
                    pltpu.async_copy(src, dst, sem)
                else:
                    pltpu.make_async_copy(src, dst, sem).wait()

        def transpose_scale(rows, trans):
            def tok4(i, carry):
                t0 = i * 4
                for dt in range(4):
                    t = t0 + dt
                    colv = jnp.full((LANES,), 0, jnp.int32) + t
                    for k, j0 in enumerate(range(0, D_OUT, LANES)):
                        v = rows[t, pl.ds(j0, LANES)] * SCALE
                        plsc.store_scatter(trans, [rowv[k], colv], v)
                return carry

            lax.fori_loop(0, BLK // 4, tok4, 0)

        fire_gather(0, rows_a, sem_a)

        def pair(g, carry):
            c0 = 2 * g
            fire_gather(c0 + 1, rows_b, sem_b)
            drain_gather(c0, rows_a, sem_a)

            @pl.when(g > 0)
            def _():
                out_tiles(c0 - 2, trans_a, sem_oa, False)

            transpose_scale(rows_a, trans_a)
            out_tiles(c0, trans_a, sem_oa, True)

            @pl.when(g + 1 < n_pair)
            def _():
                fire_gather(c0 + 2, rows_a, sem_a)

            drain_gather(c0 + 1, rows_b, sem_b)

            @pl.when(g > 0)
            def _():
                out_tiles(c0 - 1, trans_b, sem_ob, False)

            transpose_scale(rows_b, trans_b)
            out_tiles(c0 + 1, trans_b, sem_ob, True)
            return carry

        lax.fori_loop(0, n_pair, pair, 0)
        out_tiles(blocks_per_w - 2, trans_a, sem_oa, False)
        out_tiles(blocks_per_w - 1, trans_b, sem_ob, False)

    return pl.kernel(
        body,
        out_type=jax.ShapeDtypeStruct(
            (S1, D_OUT // 8, S0 // BLK, 8, BLK), jnp.float32
        ),
        mesh=mesh,
        compiler_params=pltpu.CompilerParams(
            use_tc_tiling_on_sc=False, needs_layout_passes=False
        ),
        scratch_types=[
            pltpu.VMEM((b_per_w,), jnp.int32),
            pltpu.VMEM((BLK, D_OUT), jnp.float32),
            pltpu.VMEM((BLK, D_OUT), jnp.float32),
            pltpu.VMEM((D_OUT, TPAD), jnp.float32),
            pltpu.VMEM((D_OUT, TPAD), jnp.float32),
            pltpu.SemaphoreType.DMA,
            pltpu.SemaphoreType.DMA,
            pltpu.SemaphoreType.DMA,
            pltpu.SemaphoreType.DMA,
        ],
    )


def kernel(x, table):
    s0, s1 = x.shape
    v, d = table.shape
    xt = x.T.reshape(s0 * s1).astype(jnp.int32)
    out5d = _build(s0, s1, v)(xt, table)
    return out5d.transpose(2, 4, 0, 1, 3).reshape(s0, s1, d)


# speedup vs baseline: 1.1461x; 1.0003x over previous
import functools

import jax
import jax.numpy as jnp
from jax import lax
from jax.experimental import pallas as pl
from jax.experimental.pallas import tpu as pltpu
from jax.experimental.pallas import tpu_sc as plsc

D_OUT = 64
SCALE = 8.0
LANES = 16
BLK = 128
TPAD = 129


@functools.cache
def _build(S0: int, S1: int, V: int):
    info = plsc.get_sparse_core_info()
    NC, NS = info.num_cores, info.num_subcores
    NW = NC * NS
    s0_blocks = S0 // BLK
    n_blocks = S1 * s0_blocks
    assert n_blocks % (2 * NW) == 0, (S0, S1, NW)
    blocks_per_w = n_blocks // NW
    n_pair = blocks_per_w // 2
    b_per_w = blocks_per_w * BLK
    mesh = plsc.VectorSubcoreMesh(core_axis_name="c", subcore_axis_name="s")

    def body(x_hbm, table_hbm, out_hbm, idx_v, rows_a, rows_b, rows_c, rows_d,
             trans_a, trans_b, sem_a, sem_b, sem_c, sem_d, sem_oa, sem_ob):
        wid = lax.axis_index("s") * NC + lax.axis_index("c")
        base = wid * b_per_w
        pltpu.sync_copy(x_hbm.at[pl.ds(base, b_per_w)], idx_v)
        iota16 = lax.iota(jnp.int32, 16)
        rowv = [iota16 + j0 for j0 in range(0, D_OUT, LANES)]

        def fire_gather(b, buf, sem):
            pltpu.async_copy(
                table_hbm.at[idx_v.at[pl.ds(b * BLK, BLK)]], buf, sem
            )

        def drain_gather(b, buf, sem):
            pltpu.make_async_copy(
                table_hbm.at[idx_v.at[pl.ds(b * BLK, BLK)]], buf, sem
            ).wait()

        def out_tiles(b, trans, sem, fire):
            blk_id = wid * blocks_per_w + b
            s1 = blk_id // s0_blocks
            s0b = lax.rem(blk_id, s0_blocks)
            for jb in range(D_OUT // 8):
                src = trans.at[pl.ds(jb * 8, 8), pl.ds(0, BLK)]
                dst = out_hbm.at[s1, jb, s0b]
                if fire:
                    pltpu.async_copy(src, dst, sem)
                else:
                    pltpu.make_async_copy(src, dst, sem).wait()

        def transpose_scale(rows, trans):
            def tok4(i, carry):
                t0 = i * 4
                for dt in range(4):
                    t = t0 + dt
                    colv = jnp.full((LANES,), 0, jnp.int32) + t
                    for k, j0 in enumerate(range(0, D_OUT, LANES)):
                        v = rows[t, pl.ds(j0, LANES)] * SCALE
                        plsc.store_scatter(trans, [rowv[k], colv], v)
                return carry

            lax.fori_loop(0, BLK // 4, tok4, 0)

        rows = [rows_a, rows_b, rows_c, rows_d]
        sems = [sem_a, sem_b, sem_c, sem_d]
        trans = [trans_a, trans_b]
        osems = [sem_oa, sem_ob]

        for b in range(4):
            fire_gather(b, rows[b], sems[b])

        def quad(g, carry):
            c0 = 4 * g
            for b in range(4):
                c = c0 + b
                drain_gather(c, rows[b], sems[b])
                p = b % 2

                @pl.when(c >= 2)
                def _():
                    out_tiles(c - 2, trans[p], osems[p], False)

                transpose_scale(rows[b], trans[p])
                out_tiles(c, trans[p], osems[p], True)

                @pl.when(c + 4 < blocks_per_w)
                def _():
                    fire_gather(c + 4, rows[b], sems[b])

            return carry

        lax.fori_loop(0, blocks_per_w // 4, quad, 0)
        out_tiles(blocks_per_w - 2, trans[0], osems[0], False)
        out_tiles(blocks_per_w - 1, trans[1], osems[1], False)

    return pl.kernel(
        body,
        out_type=jax.ShapeDtypeStruct(
            (S1, D_OUT // 8, S0 // BLK, 8, BLK), jnp.float32
        ),
        mesh=mesh,
        compiler_params=pltpu.CompilerParams(
            use_tc_tiling_on_sc=False, needs_layout_passes=False
        ),
        scratch_types=[
            pltpu.VMEM((b_per_w,), jnp.int32),
            pltpu.VMEM((BLK, D_OUT), jnp.float32),
            pltpu.VMEM((BLK, D_OUT), jnp.float32),
            pltpu.VMEM((BLK, D_OUT), jnp.float32),
            pltpu.VMEM((BLK, D_OUT), jnp.float32),
            pltpu.VMEM((D_OUT, TPAD), jnp.float32),
            pltpu.VMEM((D_OUT, TPAD), jnp.float32),
            pltpu.SemaphoreType.DMA,
            pltpu.SemaphoreType.DMA,
            pltpu.SemaphoreType.DMA,
            pltpu.SemaphoreType.DMA,
            pltpu.SemaphoreType.DMA,
            pltpu.SemaphoreType.DMA,
        ],
    )


def kernel(x, table):
    s0, s1 = x.shape
    v, d = table.shape
    xt = x.T.reshape(s0 * s1).astype(jnp.int32)
    out5d = _build(s0, s1, v)(xt, table)
    return out5d.transpose(2, 4, 0, 1, 3).reshape(s0, s1, d)
